# Initial kernel scaffold; baseline (speedup 1.0000x reference)
import jax
import jax.numpy as jnp
from jax import lax
from jax.experimental import pallas as pl
from jax.experimental.pallas import tpu as pltpu
from jax.experimental.pallas import tpu_sc as plsc

V = 1000000
D = 64
A = 32
B = 16384
P = 10
N = 50

NC = 2
NS = 16
NW = NC * NS
BW = B // NW
CH = 8
NCH = BW // CH

_PAD = 40.0


def _logsig(x):
    z = jnp.exp(-jnp.abs(x))
    s = z / (2.0 + z)
    s2 = s * s
    poly = 1.0 + s2 * (
        (1.0 / 3.0) + s2 * ((1.0 / 5.0) + s2 * ((1.0 / 7.0) + s2 * (1.0 / 9.0)))
    )
    return jnp.minimum(x, 0.0) - 2.0 * s * poly


def _body(il_hbm, plb_hbm, nlb_hbm, emb_hbm, kw_hbm, qw_hbm, out_hbm,
          il_v, plv, nlv, in_rows, ik_rows, pos_rows, posk_rows, neg_rows,
          out_v, sems):
    cid = lax.axis_index("c")
    sid = lax.axis_index("s")
    wid = sid * NC + cid
    base = wid * BW

    lane = lax.iota(jnp.int32, 16)

    pltpu.sync_copy(il_hbm.at[pl.ds(base, BW)], il_v)
    pltpu.sync_copy(plb_hbm.at[pl.ds(base * P, BW * P)], plv)
    pltpu.sync_copy(nlb_hbm.at[pl.ds(base * N, BW * N)], nlv)

    def chunk_copies(c, slot):
        ilo = c * CH
        cps = [
            pltpu.make_async_copy(
                emb_hbm.at[il_v.at[pl.ds(ilo, CH)]], in_rows.at[slot],
                sems.at[slot]),
            pltpu.make_async_copy(
                kw_hbm.at[il_v.at[pl.ds(ilo, CH)]], ik_rows.at[slot],
                sems.at[slot]),
            pltpu.make_async_copy(
                emb_hbm.at[plv.at[pl.ds(ilo * P, CH * P)]], pos_rows.at[slot],
                sems.at[slot]),
            pltpu.make_async_copy(
                qw_hbm.at[plv.at[pl.ds(ilo * P, CH * P)]], posk_rows.at[slot],
                sems.at[slot]),
        ]
        for j in range(5):
            cps.append(pltpu.make_async_copy(
                emb_hbm.at[nlv.at[pl.ds(ilo * N + j * 80, 80)]],
                neg_rows.at[slot, pl.ds(j * 80, 80)],
                sems.at[slot]))
        return cps

    def fire(c, slot):
        for cp in chunk_copies(c, slot):
            cp.start()

    def drain(c, slot):
        for cp in chunk_copies(c, slot):
            cp.wait()

    def compute(c, slot):
        def body_b(t, carry):
            e = [in_rows[slot, t, pl.ds(16 * j, 16)] for j in range(4)]
            kk = [ik_rows[slot, t, pl.ds(16 * j, 16)] for j in range(2)]

            pos_vec = jnp.full((16,), _PAD, jnp.float32)
            for p in range(P):
                r = t * P + p
                sv = pos_rows[slot, r, pl.ds(0, 16)] * e[0]
                for j in range(1, 4):
                    sv = sv + pos_rows[slot, r, pl.ds(16 * j, 16)] * e[j]
                s1 = jnp.sum(sv)
                kv = (posk_rows[slot, r, pl.ds(0, 16)] * kk[0]
                      + posk_rows[slot, r, pl.ds(16, 16)] * kk[1])
                s2 = jnp.sum(kv)
                pos_vec = jnp.where(lane == p, s1 * s2, pos_vec)
            acc = _logsig(pos_vec)

            for blk in range(4):
                cnt = 16 if blk < 3 else N - 48
                neg_vec = jnp.full((16,), _PAD, jnp.float32)
                for q in range(cnt):
                    r = t * N + blk * 16 + q
                    sv = neg_rows[slot, r, pl.ds(0, 16)] * e[0]
                    for j in range(1, 4):
                        sv = sv + neg_rows[slot, r, pl.ds(16 * j, 16)] * e[j]
                    neg_vec = jnp.where(lane == q, -jnp.sum(sv), neg_vec)
                acc = acc + _logsig(neg_vec)

            res = -jnp.sum(acc)
            ib = c * CH + t
            plsc.store_scatter(
                out_v,
                [jnp.full((16,), ib, jnp.int32)],
                jnp.full((16,), res, jnp.float32),
                mask=lane == 0)
            return carry
        lax.fori_loop(0, CH, body_b, 0)

    fire(0, 0)

    def gbody(g, carry):
        c0 = 2 * g
        fire(c0 + 1, 1)
        drain(c0, 0)
        compute(c0, 0)
        pl.when(c0 + 2 < NCH)(lambda: fire(c0 + 2, 0))
        drain(c0 + 1, 1)
        compute(c0 + 1, 1)
        return carry

    lax.fori_loop(0, NCH // 2, gbody, 0)

    pltpu.sync_copy(out_v, out_hbm.at[pl.ds(base, BW)])


@jax.jit
def _run(il, plb, nlb, emb, kw, qw):
    mesh = plsc.VectorSubcoreMesh(core_axis_name="c", subcore_axis_name="s")
    f = pl.kernel(
        _body,
        out_type=jax.ShapeDtypeStruct((B,), jnp.float32),
        mesh=mesh,
        scratch_types=[
            pltpu.VMEM((BW,), jnp.int32),
            pltpu.VMEM((BW * P,), jnp.int32),
            pltpu.VMEM((BW * N,), jnp.int32),
            pltpu.VMEM((2, CH, D), jnp.float32),
            pltpu.VMEM((2, CH, A), jnp.float32),
            pltpu.VMEM((2, CH * P, D), jnp.float32),
            pltpu.VMEM((2, CH * P, A), jnp.float32),
            pltpu.VMEM((2, CH * N, D), jnp.float32),
            pltpu.VMEM((BW,), jnp.float32),
            pltpu.SemaphoreType.DMA((2,)),
        ],
    )
    return f(il, plb, nlb, emb, kw, qw)


def kernel(input_labels, pos_labels, neg_labels, in_embed_w, k_w, q_w):
    il = input_labels.astype(jnp.int32)
    plb = pos_labels.reshape(-1).astype(jnp.int32)
    nlb = neg_labels.reshape(-1).astype(jnp.int32)
    return _run(il, plb, nlb, in_embed_w, k_w, q_w)


# speedup vs baseline: 3.5198x; 3.5198x over previous
import jax
import jax.numpy as jnp
from jax import lax
from jax.experimental import pallas as pl
from jax.experimental.pallas import tpu as pltpu
from jax.experimental.pallas import tpu_sc as plsc

V = 1000000
D = 64
A = 32
B = 16384
P = 10
N = 50

NC = 2
NS = 16
NW = NC * NS
BW = B // NW
CH = 8
NCH = BW // CH

_PAD = 40.0


def _logsig(x):
    z = jnp.exp(-jnp.abs(x))
    s = z / (2.0 + z)
    s2 = s * s
    poly = 1.0 + s2 * (
        (1.0 / 3.0) + s2 * ((1.0 / 5.0) + s2 * ((1.0 / 7.0) + s2 * (1.0 / 9.0)))
    )
    return jnp.minimum(x, 0.0) - 2.0 * s * poly


def _body(il_hbm, plb_hbm, nlb_hbm, emb_hbm, kw_hbm, qw_hbm, out_hbm,
          il_v, plv, nlv, in_rows, ik_rows, pos_rows, posk_rows, neg_rows,
          out_v, sems):
    cid = lax.axis_index("c")
    sid = lax.axis_index("s")
    wid = sid * NC + cid
    base = wid * BW

    lane = lax.iota(jnp.int32, 16)

    pltpu.sync_copy(il_hbm.at[pl.ds(base, BW)], il_v)
    pltpu.sync_copy(plb_hbm.at[pl.ds(base * P, BW * P)], plv)
    pltpu.sync_copy(nlb_hbm.at[pl.ds(base * N, BW * N)], nlv)

    def chunk_copies(c, slot):
        ilo = c * CH
        cps = [
            pltpu.make_async_copy(
                emb_hbm.at[il_v.at[pl.ds(ilo, CH)]], in_rows.at[slot],
                sems.at[slot]),
            pltpu.make_async_copy(
                kw_hbm.at[il_v.at[pl.ds(ilo, CH)]], ik_rows.at[slot],
                sems.at[slot]),
            pltpu.make_async_copy(
                emb_hbm.at[plv.at[pl.ds(ilo * P, CH * P)]], pos_rows.at[slot],
                sems.at[slot]),
            pltpu.make_async_copy(
                qw_hbm.at[plv.at[pl.ds(ilo * P, CH * P)]], posk_rows.at[slot],
                sems.at[slot]),
        ]
        for j in range(5):
            cps.append(pltpu.make_async_copy(
                emb_hbm.at[nlv.at[pl.ds(ilo * N + j * 80, 80)]],
                neg_rows.at[slot, pl.ds(j * 80, 80)],
                sems.at[slot]))
        return cps

    def fire(c, slot):
        for cp in chunk_copies(c, slot):
            cp.start()

    def drain(c, slot):
        for cp in chunk_copies(c, slot):
            cp.wait()

    def compute(c, slot):
        def body_b(t, carry):
            e = [in_rows[slot, t, pl.ds(16 * j, 16)] for j in range(4)]
            kk = [ik_rows[slot, t, pl.ds(16 * j, 16)] for j in range(2)]

            pos_vec = jnp.full((16,), _PAD, jnp.float32)
            for p in range(P):
                r = t * P + p
                sv = pos_rows[slot, r, pl.ds(0, 16)] * e[0]
                for j in range(1, 4):
                    sv = sv + pos_rows[slot, r, pl.ds(16 * j, 16)] * e[j]
                s1 = jnp.sum(sv)
                kv = (posk_rows[slot, r, pl.ds(0, 16)] * kk[0]
                      + posk_rows[slot, r, pl.ds(16, 16)] * kk[1])
                s2 = jnp.sum(kv)
                pos_vec = jnp.where(lane == p, s1 * s2, pos_vec)
            acc = _logsig(pos_vec)

            for blk in range(4):
                cnt = 16 if blk < 3 else N - 48
                neg_vec = jnp.full((16,), _PAD, jnp.float32)
                for q in range(cnt):
                    r = t * N + blk * 16 + q
                    sv = neg_rows[slot, r, pl.ds(0, 16)] * e[0]
                    for j in range(1, 4):
                        sv = sv + neg_rows[slot, r, pl.ds(16 * j, 16)] * e[j]
                    neg_vec = jnp.where(lane == q, -jnp.sum(sv), neg_vec)
                acc = acc + _logsig(neg_vec)

            res = -jnp.sum(acc)
            ib = c * CH + t
            plsc.store_scatter(
                out_v,
                [jnp.full((16,), ib, jnp.int32)],
                jnp.full((16,), res, jnp.float32),
                mask=lane == 0)
            return carry
        lax.fori_loop(0, CH, body_b, 0)

    fire(0, 0)

    def gbody(g, carry):
        c0 = 2 * g
        fire(c0 + 1, 1)
        drain(c0, 0)
        compute(c0, 0)
        pl.when(c0 + 2 < NCH)(lambda: fire(c0 + 2, 0))
        drain(c0 + 1, 1)
        compute(c0 + 1, 1)
        return carry

    lax.fori_loop(0, NCH // 2, gbody, 0)

    pltpu.sync_copy(out_v, out_hbm.at[pl.ds(base, BW)])


@jax.jit
def _run(il, plb, nlb, emb, kw, qw):
    mesh = plsc.VectorSubcoreMesh(core_axis_name="c", subcore_axis_name="s")
    f = pl.kernel(
        _body,
        out_type=jax.ShapeDtypeStruct((B,), jnp.float32),
        mesh=mesh,
        compiler_params=pltpu.CompilerParams(
            needs_layout_passes=False, use_tc_tiling_on_sc=False),
        scratch_types=[
            pltpu.VMEM((BW,), jnp.int32),
            pltpu.VMEM((BW * P,), jnp.int32),
            pltpu.VMEM((BW * N,), jnp.int32),
            pltpu.VMEM((2, CH, D), jnp.float32),
            pltpu.VMEM((2, CH, A), jnp.float32),
            pltpu.VMEM((2, CH * P, D), jnp.float32),
            pltpu.VMEM((2, CH * P, A), jnp.float32),
            pltpu.VMEM((2, CH * N, D), jnp.float32),
            pltpu.VMEM((BW,), jnp.float32),
            pltpu.SemaphoreType.DMA((2,)),
        ],
    )
    return f(il, plb, nlb, emb, kw, qw)


def kernel(input_labels, pos_labels, neg_labels, in_embed_w, k_w, q_w):
    il = input_labels.astype(jnp.int32)
    plb = pos_labels.reshape(-1).astype(jnp.int32)
    nlb = neg_labels.reshape(-1).astype(jnp.int32)
    return _run(il, plb, nlb, in_embed_w, k_w, q_w)


# speedup vs baseline: 3.9308x; 1.1168x over previous
import jax
import jax.numpy as jnp
from jax import lax
from jax.experimental import pallas as pl
from jax.experimental.pallas import tpu as pltpu
from jax.experimental.pallas import tpu_sc as plsc

V = 1000000
D = 64
A = 32
B = 16384
P = 10
N = 50

NC = 2
NS = 16
NW = NC * NS
BW = B // NW
CH = 4
NCH = BW // CH

_PAD = 40.0


def _logsig(x):
    z = jnp.exp(-jnp.abs(x))
    s = z / (2.0 + z)
    s2 = s * s
    poly = 1.0 + s2 * (
        (1.0 / 3.0) + s2 * ((1.0 / 5.0) + s2 * ((1.0 / 7.0) + s2 * (1.0 / 9.0)))
    )
    return jnp.minimum(x, 0.0) - 2.0 * s * poly


def _body(il_hbm, plb_hbm, nlb_hbm, emb_hbm, kw_hbm, qw_hbm, out_hbm,
          il_v, plv, nlv, in_rows, ik_rows, pos_rows, posk_rows, neg_rows,
          out_v, sems):
    cid = lax.axis_index("c")
    sid = lax.axis_index("s")
    wid = sid * NC + cid
    base = wid * BW

    lane = lax.iota(jnp.int32, 16)

    pltpu.sync_copy(il_hbm.at[pl.ds(base, BW)], il_v.at[pl.ds(0, BW)])
    pltpu.sync_copy(plb_hbm.at[pl.ds(base * P, BW * P)], plv.at[pl.ds(0, BW * P)])
    pltpu.sync_copy(nlb_hbm.at[pl.ds(base * N, BW * N)], nlv.at[pl.ds(0, BW * N)])

    def fire(c, slot):
        sem = sems.at[slot]
        lv = il_v[pl.ds(c * CH, 16)]
        for j in range(CH):
            lab = lv[j]
            pltpu.async_copy(emb_hbm.at[pl.ds(lab, 1)],
                             in_rows.at[slot, pl.ds(j, 1)], sem)
            pltpu.async_copy(kw_hbm.at[pl.ds(lab, 1)],
                             ik_rows.at[slot, pl.ds(j, 1)], sem)
        for v in range((CH * P + 15) // 16):
            pv = plv[pl.ds(c * CH * P + v * 16, 16)]
            for j in range(min(16, CH * P - v * 16)):
                lab = pv[j]
                r = v * 16 + j
                pltpu.async_copy(emb_hbm.at[pl.ds(lab, 1)],
                                 pos_rows.at[slot, pl.ds(r, 1)], sem)
                pltpu.async_copy(qw_hbm.at[pl.ds(lab, 1)],
                                 posk_rows.at[slot, pl.ds(r, 1)], sem)
        for v in range((CH * N + 15) // 16):
            nv = nlv[pl.ds(c * CH * N + v * 16, 16)]
            for j in range(min(16, CH * N - v * 16)):
                lab = nv[j]
                r = v * 16 + j
                pltpu.async_copy(emb_hbm.at[pl.ds(lab, 1)],
                                 neg_rows.at[slot, pl.ds(r, 1)], sem)

    def drain(slot):
        sem = sems.at[slot]
        pltpu.make_async_copy(
            emb_hbm.at[pl.ds(0, CH)], in_rows.at[slot], sem).wait()
        pltpu.make_async_copy(
            kw_hbm.at[pl.ds(0, CH)], ik_rows.at[slot], sem).wait()
        pltpu.make_async_copy(
            emb_hbm.at[pl.ds(0, CH * P)], pos_rows.at[slot], sem).wait()
        pltpu.make_async_copy(
            qw_hbm.at[pl.ds(0, CH * P)], posk_rows.at[slot], sem).wait()
        pltpu.make_async_copy(
            emb_hbm.at[pl.ds(0, CH * N)], neg_rows.at[slot], sem).wait()

    def compute(c, slot):
        def body_b(t, carry):
            e = [in_rows[slot, t, pl.ds(16 * j, 16)] for j in range(4)]
            kk = [ik_rows[slot, t, pl.ds(16 * j, 16)] for j in range(2)]

            pos_vec = jnp.full((16,), _PAD, jnp.float32)
            for p in range(P):
                r = t * P + p
                sv = pos_rows[slot, r, pl.ds(0, 16)] * e[0]
                for j in range(1, 4):
                    sv = sv + pos_rows[slot, r, pl.ds(16 * j, 16)] * e[j]
                s1 = jnp.sum(sv)
                kv = (posk_rows[slot, r, pl.ds(0, 16)] * kk[0]
                      + posk_rows[slot, r, pl.ds(16, 16)] * kk[1])
                s2 = jnp.sum(kv)
                pos_vec = jnp.where(lane == p, s1 * s2, pos_vec)
            acc = _logsig(pos_vec)

            for blk in range(4):
                cnt = 16 if blk < 3 else N - 48
                neg_vec = jnp.full((16,), _PAD, jnp.float32)
                for q in range(cnt):
                    r = t * N + blk * 16 + q
                    sv = neg_rows[slot, r, pl.ds(0, 16)] * e[0]
                    for j in range(1, 4):
                        sv = sv + neg_rows[slot, r, pl.ds(16 * j, 16)] * e[j]
                    neg_vec = jnp.where(lane == q, -jnp.sum(sv), neg_vec)
                acc = acc + _logsig(neg_vec)

            res = -jnp.sum(acc)
            ib = c * CH + t
            plsc.store_scatter(
                out_v,
                [jnp.full((16,), ib, jnp.int32)],
                jnp.full((16,), res, jnp.float32),
                mask=lane == 0)
            return carry
        lax.fori_loop(0, CH, body_b, 0)

    fire(0, 0)

    def gbody(g, carry):
        c0 = 2 * g
        fire(c0 + 1, 1)
        drain(0)
        compute(c0, 0)
        pl.when(c0 + 2 < NCH)(lambda: fire(c0 + 2, 0))
        drain(1)
        compute(c0 + 1, 1)
        return carry

    lax.fori_loop(0, NCH // 2, gbody, 0)

    pltpu.sync_copy(out_v, out_hbm.at[pl.ds(base, BW)])


@jax.jit
def _run(il, plb, nlb, emb, kw, qw):
    mesh = plsc.VectorSubcoreMesh(core_axis_name="c", subcore_axis_name="s")
    f = pl.kernel(
        _body,
        out_type=jax.ShapeDtypeStruct((B,), jnp.float32),
        mesh=mesh,
        compiler_params=pltpu.CompilerParams(
            needs_layout_passes=False, use_tc_tiling_on_sc=True),
        scratch_types=[
            pltpu.VMEM((BW + 16,), jnp.int32),
            pltpu.VMEM((BW * P + 16,), jnp.int32),
            pltpu.VMEM((BW * N + 16,), jnp.int32),
            pltpu.VMEM((2, CH, D), jnp.float32),
            pltpu.VMEM((2, CH, A), jnp.float32),
            pltpu.VMEM((2, CH * P, D), jnp.float32),
            pltpu.VMEM((2, CH * P, A), jnp.float32),
            pltpu.VMEM((2, CH * N, D), jnp.float32),
            pltpu.VMEM((BW,), jnp.float32),
            pltpu.SemaphoreType.DMA((2,)),
        ],
    )
    return f(il, plb, nlb, emb, kw, qw)


def kernel(input_labels, pos_labels, neg_labels, in_embed_w, k_w, q_w):
    il = input_labels.astype(jnp.int32)
    plb = pos_labels.reshape(-1).astype(jnp.int32)
    nlb = neg_labels.reshape(-1).astype(jnp.int32)
    return _run(il, plb, nlb, in_embed_w, k_w, q_w)
